# baseline (speedup 1.0000x reference)
import functools

import jax
import jax.numpy as jnp
from jax import lax
from jax.experimental import pallas as pl
from jax.experimental.pallas import tpu as pltpu
import jax.experimental.pallas.tpu_sc as plsc

D_MODEL = 1024
D_FFN = 2816
NEXP = 16
NTOK = 2048
NPAIR = 2 * NTOK
RBLK = 128
NROWS = NPAIR + NEXP * RBLK
NBLK = NROWS // RBLK
NJ = 2
FJ = D_FFN // NJ

NC, NS = 2, 16
NW = NC * NS
_MESH = dict(core_axis_name="c", subcore_axis_name="s", num_cores=NC,
             num_subcores=NS)


def _route_body(x_ref, wr_ref, br_ref, s0_ref, s1_ref, g0_ref, g1_ref, blk_ref):
    f32 = jnp.float32
    x = x_ref[...]
    logits = lax.dot_general(x, wr_ref[...], (((1,), (1,)), ((), ())),
                             preferred_element_type=f32) + br_ref[...]
    idx = lax.broadcasted_iota(jnp.int32, (NTOK, NEXP), 1)
    m1 = jnp.max(logits, 1, keepdims=True)
    i1 = jnp.min(jnp.where(logits == m1, idx, NEXP), 1, keepdims=True)
    oh0 = idx == i1
    l2 = jnp.where(oh0, -jnp.inf, logits)
    m2 = jnp.max(l2, 1, keepdims=True)
    i2 = jnp.min(jnp.where(l2 == m2, idx, NEXP), 1, keepdims=True)
    oh1 = idx == i2
    e = jnp.exp(m2 - m1)
    g0_ref[...] = jnp.broadcast_to(1.0 / (1.0 + e), (NTOK, NEXP))
    g1_ref[...] = jnp.broadcast_to(e / (1.0 + e), (NTOK, NEXP))

    ohf0 = oh0.astype(f32)
    ohf1 = oh1.astype(f32)
    cnts = ohf0 + ohf1
    CH = 256
    ii = lax.broadcasted_iota(jnp.int32, (CH, CH), 0)
    jj = lax.broadcasted_iota(jnp.int32, (CH, CH), 1)
    ltri = (ii > jj).astype(f32)
    run = jnp.zeros((1, NEXP), f32)
    segs = []
    for c in range(NTOK // CH):
        seg = cnts[c * CH:(c + 1) * CH]
        segs.append(lax.dot_general(ltri, seg, (((1,), (0,)), ((), ())),
                                    preferred_element_type=f32) + run)
        run = run + jnp.sum(seg, 0, keepdims=True)
    cum = jnp.concatenate(segs, 0)

    padded = jnp.ceil(run / RBLK) * RBLK
    iu = lax.broadcasted_iota(jnp.int32, (NEXP, NEXP), 0)
    ju = lax.broadcasted_iota(jnp.int32, (NEXP, NEXP), 1)
    utri = (iu < ju).astype(f32)
    off = lax.dot_general(padded, utri, (((1,), (0,)), ((), ())),
                          preferred_element_type=f32)
    rank0 = jnp.sum(cum * ohf0, 1, keepdims=True)
    rank1 = jnp.sum(cum * ohf1, 1, keepdims=True)
    s0_ref[...] = (jnp.sum(off * ohf0, 1, keepdims=True) + rank0).astype(jnp.int32)
    s1_ref[...] = (jnp.sum(off * ohf1, 1, keepdims=True) + rank1).astype(jnp.int32)

    total = jnp.sum(padded)
    bi = lax.broadcasted_iota(jnp.int32, (NBLK, NEXP), 0)
    ci = lax.broadcasted_iota(jnp.int32, (NBLK, NEXP), 1)
    beff = jnp.minimum((bi * RBLK).astype(f32), total - RBLK)
    hit = (jnp.broadcast_to(off, (NBLK, NEXP)) <= beff) & (ci >= 1)
    nused = jnp.broadcast_to(total / RBLK, (1, 1)).astype(jnp.int32)
    blk_ref[...] = jnp.concatenate(
        [jnp.sum(hit.astype(jnp.int32), 1, keepdims=True), nused], axis=0)


_route = pl.pallas_call(
    _route_body,
    out_shape=[
        jax.ShapeDtypeStruct((NTOK, 1), jnp.int32),
        jax.ShapeDtypeStruct((NTOK, 1), jnp.int32),
        jax.ShapeDtypeStruct((NTOK, NEXP), jnp.float32),
        jax.ShapeDtypeStruct((NTOK, NEXP), jnp.float32),
        jax.ShapeDtypeStruct((NBLK + 1, 1), jnp.int32),
    ],
)


_D_CH = 32
_D_PW = NPAIR // NW


def _dispatch_body(tok_hbm, slot_hbm, x_hbm, xs_hbm, tok_v, slot_v, buf_v, sem):
    wid = lax.axis_index("s") * NC + lax.axis_index("c")
    base = wid * _D_PW
    for cc in range(_D_PW // _D_CH):
        o = base + cc * _D_CH
        pltpu.sync_copy(tok_hbm.at[pl.ds(o, _D_CH)], tok_v)
        pltpu.sync_copy(slot_hbm.at[pl.ds(o, _D_CH)], slot_v)
        pltpu.async_copy(x_hbm.at[tok_v], buf_v, sem).wait()
        pltpu.async_copy(buf_v, xs_hbm.at[slot_v], sem).wait()


@functools.cache
def _dispatch():
    return pl.kernel(
        _dispatch_body,
        out_type=jax.ShapeDtypeStruct((NROWS, D_MODEL), jnp.float32),
        mesh=plsc.VectorSubcoreMesh(**_MESH),
        scratch_types=[
            pltpu.VMEM((_D_CH,), jnp.int32),
            pltpu.VMEM((_D_CH,), jnp.int32),
            pltpu.VMEM((_D_CH, D_MODEL), jnp.float32),
            pltpu.SemaphoreType.DMA,
        ],
    )


def _ffn_body(blk_ref, xs_ref, w1_ref, w3_ref, w2_ref, out0_ref, out1_ref,
              w1b_s, w3b_s, w2b_s):
    f32 = jnp.float32
    bf16 = jnp.bfloat16
    j = pl.program_id(0)
    b = pl.program_id(1)
    real = b < blk_ref[NBLK]
    prev = blk_ref[jnp.maximum(b - 1, 0)]
    boundary = jnp.logical_or(b == 0, blk_ref[b] != prev)

    @pl.when(jnp.logical_and(real, boundary))
    def _():
        w1b_s[...] = w1_ref[0].astype(bf16)
        w3b_s[...] = w3_ref[0].astype(bf16)
        w2b_s[...] = w2_ref[0].astype(bf16)

    @pl.when(real)
    def _():
        xb = xs_ref[...].astype(bf16)
        a = lax.dot_general(xb, w1b_s[...], (((1,), (1,)), ((), ())),
                            preferred_element_type=f32)
        c = lax.dot_general(xb, w3b_s[...], (((1,), (1,)), ((), ())),
                            preferred_element_type=f32)
        h = a * (1.0 / (1.0 + jnp.exp(-a))) * c
        p = lax.dot_general(h.astype(bf16), w2b_s[...], (((1,), (1,)), ((), ())),
                            preferred_element_type=f32)

        @pl.when(j == 0)
        def _():
            out0_ref[...] = p

        @pl.when(j != 0)
        def _():
            out1_ref[...] = p


_ffn = pl.pallas_call(
    _ffn_body,
    grid_spec=pltpu.PrefetchScalarGridSpec(
        num_scalar_prefetch=1,
        grid=(NJ, NBLK),
        in_specs=[
            pl.BlockSpec((RBLK, D_MODEL), lambda j, b, blk: (b, 0)),
            pl.BlockSpec((1, FJ, D_MODEL), lambda j, b, blk: (blk[b], j, 0)),
            pl.BlockSpec((1, FJ, D_MODEL), lambda j, b, blk: (blk[b], j, 0)),
            pl.BlockSpec((1, D_MODEL, FJ), lambda j, b, blk: (blk[b], 0, j)),
        ],
        out_specs=[
            pl.BlockSpec((RBLK, D_MODEL),
                         lambda j, b, blk: (jnp.where(j == 0, b, NBLK), 0)),
            pl.BlockSpec((RBLK, D_MODEL),
                         lambda j, b, blk: (jnp.where(j == 0, NBLK, b), 0)),
        ],
        scratch_shapes=[
            pltpu.VMEM((FJ, D_MODEL), jnp.bfloat16),
            pltpu.VMEM((FJ, D_MODEL), jnp.bfloat16),
            pltpu.VMEM((D_MODEL, FJ), jnp.bfloat16),
        ],
    ),
    out_shape=[
        jax.ShapeDtypeStruct(((NBLK + 1) * RBLK, D_MODEL), jnp.float32),
        jax.ShapeDtypeStruct(((NBLK + 1) * RBLK, D_MODEL), jnp.float32),
    ],
    compiler_params=pltpu.CompilerParams(
        dimension_semantics=("arbitrary", "arbitrary")),
)


_C_CH = 16
_C_PW = NTOK // NW


def _combine_body(s0_hbm, s1_hbm, g0_hbm, g1_hbm, ysa_hbm, ysb_hbm, fin_hbm,
                  i0_v, i1_v, y0a_v, y0b_v, y1a_v, y1b_v, f_v, g0_v, g1_v,
                  sem):
    wid = lax.axis_index("s") * NC + lax.axis_index("c")
    base = wid * _C_PW
    for cc in range(_C_PW // _C_CH):
        o = base + cc * _C_CH
        pltpu.sync_copy(s0_hbm.at[pl.ds(o, _C_CH)], i0_v)
        pltpu.sync_copy(s1_hbm.at[pl.ds(o, _C_CH)], i1_v)
        pltpu.sync_copy(g0_hbm.at[pl.ds(o, _C_CH)], g0_v)
        pltpu.sync_copy(g1_hbm.at[pl.ds(o, _C_CH)], g1_v)
        cps = [pltpu.async_copy(ysa_hbm.at[i0_v], y0a_v, sem),
               pltpu.async_copy(ysb_hbm.at[i0_v], y0b_v, sem),
               pltpu.async_copy(ysa_hbm.at[i1_v], y1a_v, sem),
               pltpu.async_copy(ysb_hbm.at[i1_v], y1b_v, sem)]
        for cp in cps:
            cp.wait()
        for t in range(_C_CH):
            gv0 = g0_v[t, :]
            gv1 = g1_v[t, :]

            def body(h, carry):
                sl = pl.ds(h * 16, 16)
                f_v[t, sl] = (gv0 * (y0a_v[t, sl] + y0b_v[t, sl])
                              + gv1 * (y1a_v[t, sl] + y1b_v[t, sl]))
                return carry

            lax.fori_loop(0, D_MODEL // 16, body, 0)
        pltpu.sync_copy(f_v, fin_hbm.at[pl.ds(o, _C_CH)])


@functools.cache
def _combine():
    return pl.kernel(
        _combine_body,
        out_type=jax.ShapeDtypeStruct((NTOK, D_MODEL), jnp.float32),
        mesh=plsc.VectorSubcoreMesh(**_MESH),
        scratch_types=[
            pltpu.VMEM((_C_CH,), jnp.int32),
            pltpu.VMEM((_C_CH,), jnp.int32),
            pltpu.VMEM((_C_CH, D_MODEL), jnp.float32),
            pltpu.VMEM((_C_CH, D_MODEL), jnp.float32),
            pltpu.VMEM((_C_CH, D_MODEL), jnp.float32),
            pltpu.VMEM((_C_CH, D_MODEL), jnp.float32),
            pltpu.VMEM((_C_CH, D_MODEL), jnp.float32),
            pltpu.VMEM((_C_CH, NEXP), jnp.float32),
            pltpu.VMEM((_C_CH, NEXP), jnp.float32),
            pltpu.SemaphoreType.DMA,
        ],
    )


def kernel(x, Wr, br, Wn, bn, w1, w2, w3):
    fx = x.reshape(NTOK, D_MODEL)
    s0, s1, g0b, g1b, blk = _route(fx, Wr, br.reshape(1, NEXP))
    tok = jnp.concatenate([jnp.arange(NTOK, dtype=jnp.int32)] * 2)
    slot = jnp.concatenate([s0.reshape(-1), s1.reshape(-1)])
    xs = _dispatch()(tok, slot, fx)
    ysa, ysb = _ffn(blk.reshape(-1), xs, w1, w3, w2)
    fin = _combine()(s0.reshape(-1), s1.reshape(-1), g0b, g1b, ysa, ysb)
    return fin.reshape(x.shape)


# speedup vs baseline: 1.0790x; 1.0790x over previous
import functools

import jax
import jax.numpy as jnp
from jax import lax
from jax.experimental import pallas as pl
from jax.experimental.pallas import tpu as pltpu
import jax.experimental.pallas.tpu_sc as plsc

D_MODEL = 1024
D_FFN = 2816
NEXP = 16
NTOK = 2048
NPAIR = 2 * NTOK
RBLK = 128
NROWS = NPAIR + NEXP * RBLK
NBLK = NROWS // RBLK
NJ = 2
FJ = D_FFN // NJ

NC, NS = 2, 16
NW = NC * NS
_MESH = dict(core_axis_name="c", subcore_axis_name="s", num_cores=NC,
             num_subcores=NS)


def _route_body(x_ref, wr_ref, br_ref, s_ref, g0_ref, blk_ref):
    f32 = jnp.float32
    x = x_ref[...]
    logits = lax.dot_general(x, wr_ref[...], (((1,), (1,)), ((), ())),
                             preferred_element_type=f32) + br_ref[...]
    idx = lax.broadcasted_iota(jnp.int32, (NTOK, NEXP), 1)
    m1 = jnp.max(logits, 1, keepdims=True)
    i1 = jnp.min(jnp.where(logits == m1, idx, NEXP), 1, keepdims=True)
    oh0 = idx == i1
    l2 = jnp.where(oh0, -jnp.inf, logits)
    m2 = jnp.max(l2, 1, keepdims=True)
    i2 = jnp.min(jnp.where(l2 == m2, idx, NEXP), 1, keepdims=True)
    oh1 = idx == i2
    e = jnp.exp(m2 - m1)
    g0_ref[...] = jnp.broadcast_to(1.0 / (1.0 + e), (NTOK, NEXP))

    ohf0 = oh0.astype(f32)
    ohf1 = oh1.astype(f32)
    cnts = ohf0 + ohf1
    CH = 256
    ii = lax.broadcasted_iota(jnp.int32, (CH, CH), 0)
    jj = lax.broadcasted_iota(jnp.int32, (CH, CH), 1)
    ltri = (ii > jj).astype(f32)
    run = jnp.zeros((1, NEXP), f32)
    segs = []
    for c in range(NTOK // CH):
        seg = cnts[c * CH:(c + 1) * CH]
        segs.append(lax.dot_general(ltri, seg, (((1,), (0,)), ((), ())),
                                    preferred_element_type=f32) + run)
        run = run + jnp.sum(seg, 0, keepdims=True)
    cum = jnp.concatenate(segs, 0)

    padded = jnp.ceil(run / RBLK) * RBLK
    iu = lax.broadcasted_iota(jnp.int32, (NEXP, NEXP), 0)
    ju = lax.broadcasted_iota(jnp.int32, (NEXP, NEXP), 1)
    utri = (iu < ju).astype(f32)
    off = lax.dot_general(padded, utri, (((1,), (0,)), ((), ())),
                          preferred_element_type=f32)
    rank0 = jnp.sum(cum * ohf0, 1, keepdims=True)
    rank1 = jnp.sum(cum * ohf1, 1, keepdims=True)
    s_ref[0:NTOK] = (jnp.sum(off * ohf0, 1, keepdims=True) + rank0).astype(jnp.int32)
    s_ref[NTOK:NPAIR] = (jnp.sum(off * ohf1, 1, keepdims=True)
                         + rank1).astype(jnp.int32)

    total = jnp.sum(padded)
    bi = lax.broadcasted_iota(jnp.int32, (NBLK, NEXP), 0)
    ci = lax.broadcasted_iota(jnp.int32, (NBLK, NEXP), 1)
    beff = jnp.minimum((bi * RBLK).astype(f32), total - RBLK)
    hit = (jnp.broadcast_to(off, (NBLK, NEXP)) <= beff) & (ci >= 1)
    nused = jnp.broadcast_to(total / RBLK, (1, 1)).astype(jnp.int32)
    blk_ref[...] = jnp.concatenate(
        [jnp.sum(hit.astype(jnp.int32), 1, keepdims=True), nused], axis=0)


_route = pl.pallas_call(
    _route_body,
    out_shape=[
        jax.ShapeDtypeStruct((NPAIR, 1), jnp.int32),
        jax.ShapeDtypeStruct((NTOK, NEXP), jnp.float32),
        jax.ShapeDtypeStruct((NBLK + 1, 1), jnp.int32),
    ],
)


_D_CH = 32
_D_TW = NTOK // NW


def _dispatch_body(slot_hbm, x_hbm, xs_hbm, s0_v, s1_v, buf_v, sem):
    wid = lax.axis_index("s") * NC + lax.axis_index("c")
    base = wid * _D_TW
    for cc in range(_D_TW // _D_CH):
        o = base + cc * _D_CH
        pltpu.sync_copy(x_hbm.at[pl.ds(o, _D_CH)], buf_v)
        pltpu.sync_copy(slot_hbm.at[pl.ds(o, _D_CH)], s0_v)
        pltpu.sync_copy(slot_hbm.at[pl.ds(NTOK + o, _D_CH)], s1_v)
        c0 = pltpu.async_copy(buf_v, xs_hbm.at[s0_v], sem)
        c1 = pltpu.async_copy(buf_v, xs_hbm.at[s1_v], sem)
        c0.wait()
        c1.wait()


@functools.cache
def _dispatch():
    return pl.kernel(
        _dispatch_body,
        out_type=jax.ShapeDtypeStruct((NROWS, D_MODEL), jnp.float32),
        mesh=plsc.VectorSubcoreMesh(**_MESH),
        scratch_types=[
            pltpu.VMEM((_D_CH,), jnp.int32),
            pltpu.VMEM((_D_CH,), jnp.int32),
            pltpu.VMEM((_D_CH, D_MODEL), jnp.float32),
            pltpu.SemaphoreType.DMA,
        ],
    )


def _ffn_body(blk_ref, xs_ref, w1_ref, w3_ref, w2_ref, out0_ref, out1_ref):
    f32 = jnp.float32
    j = pl.program_id(0)
    b = pl.program_id(1)

    @pl.when(b < blk_ref[NBLK])
    def _():
        xb = xs_ref[...]
        a = lax.dot_general(xb, w1_ref[0], (((1,), (1,)), ((), ())),
                            preferred_element_type=f32)
        c = lax.dot_general(xb, w3_ref[0], (((1,), (1,)), ((), ())),
                            preferred_element_type=f32)
        h = a * (1.0 / (1.0 + jnp.exp(-a))) * c
        p = lax.dot_general(h, w2_ref[0], (((1,), (1,)), ((), ())),
                            preferred_element_type=f32)

        @pl.when(j == 0)
        def _():
            out0_ref[...] = p

        @pl.when(j != 0)
        def _():
            out1_ref[...] = p


_ffn = pl.pallas_call(
    _ffn_body,
    grid_spec=pltpu.PrefetchScalarGridSpec(
        num_scalar_prefetch=1,
        grid=(NJ, NBLK),
        in_specs=[
            pl.BlockSpec((RBLK, D_MODEL), lambda j, b, blk: (b, 0)),
            pl.BlockSpec((1, FJ, D_MODEL), lambda j, b, blk: (blk[b], j, 0)),
            pl.BlockSpec((1, FJ, D_MODEL), lambda j, b, blk: (blk[b], j, 0)),
            pl.BlockSpec((1, D_MODEL, FJ), lambda j, b, blk: (blk[b], 0, j)),
        ],
        out_specs=[
            pl.BlockSpec((RBLK, D_MODEL),
                         lambda j, b, blk: (jnp.where(j == 0, b, NBLK), 0)),
            pl.BlockSpec((RBLK, D_MODEL),
                         lambda j, b, blk: (jnp.where(j == 0, NBLK, b), 0)),
        ],
    ),
    out_shape=[
        jax.ShapeDtypeStruct(((NBLK + 1) * RBLK, D_MODEL), jnp.float32),
        jax.ShapeDtypeStruct(((NBLK + 1) * RBLK, D_MODEL), jnp.float32),
    ],
    compiler_params=pltpu.CompilerParams(
        dimension_semantics=("arbitrary", "arbitrary")),
)


_C_CH = 16
_C_PW = NTOK // NW


def _combine_body(slot_hbm, g0_hbm, ysa_hbm, ysb_hbm, fin_hbm,
                  i0_v, i1_v, y0a_v, y0b_v, y1a_v, y1b_v, f_v, g0_v, sem):
    wid = lax.axis_index("s") * NC + lax.axis_index("c")
    base = wid * _C_PW
    for cc in range(_C_PW // _C_CH):
        o = base + cc * _C_CH
        pltpu.sync_copy(slot_hbm.at[pl.ds(o, _C_CH)], i0_v)
        pltpu.sync_copy(slot_hbm.at[pl.ds(NTOK + o, _C_CH)], i1_v)
        pltpu.sync_copy(g0_hbm.at[pl.ds(o, _C_CH)], g0_v)
        cps = [pltpu.async_copy(ysa_hbm.at[i0_v], y0a_v, sem),
               pltpu.async_copy(ysb_hbm.at[i0_v], y0b_v, sem),
               pltpu.async_copy(ysa_hbm.at[i1_v], y1a_v, sem),
               pltpu.async_copy(ysb_hbm.at[i1_v], y1b_v, sem)]
        for cp in cps:
            cp.wait()
        for t in range(_C_CH):
            gv0 = g0_v[t, :]
            gv1 = 1.0 - gv0

            def body(h, carry):
                sl = pl.ds(h * 16, 16)
                f_v[t, sl] = (gv0 * (y0a_v[t, sl] + y0b_v[t, sl])
                              + gv1 * (y1a_v[t, sl] + y1b_v[t, sl]))
                return carry

            lax.fori_loop(0, D_MODEL // 16, body, 0)
        pltpu.sync_copy(f_v, fin_hbm.at[pl.ds(o, _C_CH)])


@functools.cache
def _combine():
    return pl.kernel(
        _combine_body,
        out_type=jax.ShapeDtypeStruct((NTOK, D_MODEL), jnp.float32),
        mesh=plsc.VectorSubcoreMesh(**_MESH),
        scratch_types=[
            pltpu.VMEM((_C_CH,), jnp.int32),
            pltpu.VMEM((_C_CH,), jnp.int32),
            pltpu.VMEM((_C_CH, D_MODEL), jnp.float32),
            pltpu.VMEM((_C_CH, D_MODEL), jnp.float32),
            pltpu.VMEM((_C_CH, D_MODEL), jnp.float32),
            pltpu.VMEM((_C_CH, D_MODEL), jnp.float32),
            pltpu.VMEM((_C_CH, D_MODEL), jnp.float32),
            pltpu.VMEM((_C_CH, NEXP), jnp.float32),
            pltpu.SemaphoreType.DMA,
        ],
    )


def kernel(x, Wr, br, Wn, bn, w1, w2, w3):
    fx = x.reshape(NTOK, D_MODEL)
    slot, g0b, blk = _route(fx, Wr, br.reshape(1, NEXP))
    slot = slot.reshape(-1)
    xs = _dispatch()(slot, fx)
    ysa, ysb = _ffn(blk.reshape(-1), xs, w1, w3, w2)
    fin = _combine()(slot, g0b, ysa, ysb)
    return fin.reshape(x.shape)
